# baseline (speedup 1.0000x reference)
import functools
import jax
import jax.numpy as jnp
from jax import lax
from jax.experimental import pallas as pl
from jax.experimental.pallas import tpu as pltpu
from jax.experimental.pallas import tpu_sc as plsc

_MAXV = 11
_C = 3
_D = 128
_B, _H, _W = 128, 32, 32
_P = _H * _W
_NC, _NS = 2, 16
_NW = _NC * _NS
_BPT = _B // _NW
_CPX = 128
_NCH = _BPT * _P // _CPX
_NSLOT = 4
_CTR = _MAXV * _D


def _sc_body(in_hbm, tab_hbm, out_hbm,
             inbuf, tabv, ctbuf, karr, ct_sh, rb0, rb1, rb2, rb3,
             insem, ctsem, gsem0, gsem1, gsem2, gsem3,
             osem0, osem1, osem2, osem3):
    core = lax.axis_index("c")
    sub = lax.axis_index("s")
    wid = sub * _NC + core
    b0 = wid * _BPT

    pltpu.make_async_copy(in_hbm.at[pl.ds(b0 * _C * _P, _BPT * _C * _P)],
                          inbuf, insem).start()
    pltpu.sync_copy(tab_hbm, tabv)

    @pl.when(sub < _MAXV)
    def _build_ct():
        r2 = [tabv[pl.ds((2 * _MAXV + sub) * _D + d0 * 16, 16)]
              for d0 in range(8)]
        for v1 in range(_MAXV):
            r1 = [tabv[pl.ds((_MAXV + v1) * _D + d0 * 16, 16)]
                  for d0 in range(8)]

            def ct_row(v0, _, r1=r1, v1=v1):
                for d0 in range(8):
                    ctbuf[v1 * _MAXV + v0, pl.ds(d0 * 16, 16)] = (
                        tabv[pl.ds(v0 * _D + d0 * 16, 16)] + r1[d0] + r2[d0])
                return 0

            lax.fori_loop(0, _MAXV, ct_row, 0)
        pltpu.make_async_copy(
            ctbuf, ct_sh.at[pl.ds(sub * _D, _D)], ctsem).start()

    pltpu.make_async_copy(in_hbm.at[pl.ds(b0 * _C * _P, _BPT * _C * _P)],
                          inbuf, insem).wait()

    def key_chunk(i, _):
        lb = i // (_P // 16)
        ch = i % (_P // 16)
        base = lb * _C * _P + ch * 16
        v0 = inbuf[pl.ds(base, 16)]
        v1 = inbuf[pl.ds(base + _P, 16)]
        v2 = inbuf[pl.ds(base + 2 * _P, 16)]
        karr[pl.ds(lb * _P + ch * 16, 16)] = v0 + v1 * _MAXV + v2 * _D
        return 0

    lax.fori_loop(0, _BPT * (_P // 16), key_chunk, 0)

    @pl.when(sub < _MAXV)
    def _wait_ct():
        pltpu.make_async_copy(
            ctbuf, ct_sh.at[pl.ds(sub * _D, _D)], ctsem).wait()

    plsc.subcore_barrier()

    rbs = (rb0, rb1, rb2, rb3)
    gsems = (gsem0, gsem1, gsem2, gsem3)
    osems = (osem0, osem1, osem2, osem3)
    src = ct_sh

    def gather(ci, s):
        pltpu.make_async_copy(
            src.at[karr.at[pl.ds(ci * _CPX, _CPX)]], rbs[s], gsems[s]).start()

    def gather_wait(ci, s):
        pltpu.make_async_copy(
            src.at[karr.at[pl.ds(ci * _CPX, _CPX)]], rbs[s], gsems[s]).wait()

    def out_start(ci, s):
        pltpu.make_async_copy(
            rbs[s], out_hbm.at[pl.ds(b0 * _P + ci * _CPX, _CPX)],
            osems[s]).start()

    def out_wait(ci, s):
        pltpu.make_async_copy(
            rbs[s], out_hbm.at[pl.ds(b0 * _P + ci * _CPX, _CPX)],
            osems[s]).wait()

    for s in range(_NSLOT):
        gather(s, s)

    def ring(g, _):
        for s in range(_NSLOT):
            ci = g * _NSLOT + s
            gather_wait(ci, s)
            out_start(ci, s)

            @pl.when(g < _NCH // _NSLOT - 1)
            def _next():
                out_wait(ci, s)
                gather(ci + _NSLOT, s)
        return 0

    lax.fori_loop(0, _NCH // _NSLOT, ring, 0)
    for s in range(_NSLOT):
        out_wait(_NCH - _NSLOT + s, s)


def kernel(inputs, table):
    in_flat = inputs.reshape(-1)
    tab_flat = table.reshape(-1)

    mesh = plsc.VectorSubcoreMesh(core_axis_name="c", subcore_axis_name="s")
    f = functools.partial(
        pl.kernel,
        mesh=mesh,
        out_type=jax.ShapeDtypeStruct((_B * _P, _D), jnp.float32),
        scratch_types=[
            pltpu.VMEM((_BPT * _C * _P,), jnp.int32),
            pltpu.VMEM(((_C * _MAXV) * _D,), jnp.float32),
            pltpu.VMEM((_D, _D), jnp.float32),
            pltpu.VMEM((_BPT * _P,), jnp.int32),
            pltpu.VMEM_SHARED((_CTR, _D), jnp.float32),
            pltpu.VMEM((_CPX, _D), jnp.float32),
            pltpu.VMEM((_CPX, _D), jnp.float32),
            pltpu.VMEM((_CPX, _D), jnp.float32),
            pltpu.VMEM((_CPX, _D), jnp.float32),
        ] + [pltpu.SemaphoreType.DMA] * 10,
        compiler_params=pltpu.CompilerParams(needs_layout_passes=False),
    )(_sc_body)
    out = f(in_flat, tab_flat)
    return out.reshape(_B, _H, _W, _D).transpose(0, 3, 1, 2)


# speedup vs baseline: 1.0043x; 1.0043x over previous
import functools
import jax
import jax.numpy as jnp
from jax import lax
from jax.experimental import pallas as pl
from jax.experimental.pallas import tpu as pltpu
from jax.experimental.pallas import tpu_sc as plsc

_MAXV = 11
_C = 3
_D = 128
_B, _H, _W = 128, 32, 32
_P = _H * _W
_NC, _NS = 2, 16
_NW = _NC * _NS
_BPT = _B // _NW
_CPX = 64
_NCH = _BPT * _P // _CPX
_NSLOT = 8
_CTR = _MAXV * _D


def _sc_body(in_hbm, tab_hbm, out_hbm,
             inbuf, tabv, ctbuf, karr, ct_sh,
             rb0, rb1, rb2, rb3, rb4, rb5, rb6, rb7,
             insem, ctsem, gsem0, gsem1, gsem2, gsem3,
             gsem4, gsem5, gsem6, gsem7,
             osem0, osem1, osem2, osem3, osem4, osem5, osem6, osem7):
    core = lax.axis_index("c")
    sub = lax.axis_index("s")
    wid = sub * _NC + core
    b0 = wid * _BPT

    pltpu.make_async_copy(in_hbm.at[pl.ds(b0 * _C * _P, _BPT * _C * _P)],
                          inbuf, insem).start()
    pltpu.sync_copy(tab_hbm, tabv)

    @pl.when(sub < _MAXV)
    def _build_ct():
        r2 = [tabv[pl.ds((2 * _MAXV + sub) * _D + d0 * 16, 16)]
              for d0 in range(8)]
        for v1 in range(_MAXV):
            r1 = [tabv[pl.ds((_MAXV + v1) * _D + d0 * 16, 16)]
                  for d0 in range(8)]

            def ct_row(v0, _, r1=r1, v1=v1):
                for d0 in range(8):
                    ctbuf[v1 * _MAXV + v0, pl.ds(d0 * 16, 16)] = (
                        tabv[pl.ds(v0 * _D + d0 * 16, 16)] + r1[d0] + r2[d0])
                return 0

            lax.fori_loop(0, _MAXV, ct_row, 0)
        pltpu.make_async_copy(
            ctbuf, ct_sh.at[pl.ds(sub * _D, _D)], ctsem).start()

    pltpu.make_async_copy(in_hbm.at[pl.ds(b0 * _C * _P, _BPT * _C * _P)],
                          inbuf, insem).wait()

    def key_chunk(i, _):
        lb = i // (_P // 16)
        ch = i % (_P // 16)
        base = lb * _C * _P + ch * 16
        v0 = inbuf[pl.ds(base, 16)]
        v1 = inbuf[pl.ds(base + _P, 16)]
        v2 = inbuf[pl.ds(base + 2 * _P, 16)]
        karr[pl.ds(lb * _P + ch * 16, 16)] = v0 + v1 * _MAXV + v2 * _D
        return 0

    lax.fori_loop(0, _BPT * (_P // 16), key_chunk, 0)

    @pl.when(sub < _MAXV)
    def _wait_ct():
        pltpu.make_async_copy(
            ctbuf, ct_sh.at[pl.ds(sub * _D, _D)], ctsem).wait()

    plsc.subcore_barrier()

    rbs = (rb0, rb1, rb2, rb3, rb4, rb5, rb6, rb7)
    gsems = (gsem0, gsem1, gsem2, gsem3, gsem4, gsem5, gsem6, gsem7)
    osems = (osem0, osem1, osem2, osem3, osem4, osem5, osem6, osem7)
    src = ct_sh

    def gather(ci, s):
        pltpu.make_async_copy(
            src.at[karr.at[pl.ds(ci * _CPX, _CPX)]], rbs[s], gsems[s]).start()

    def gather_wait(ci, s):
        pltpu.make_async_copy(
            src.at[karr.at[pl.ds(ci * _CPX, _CPX)]], rbs[s], gsems[s]).wait()

    def out_start(ci, s):
        pltpu.make_async_copy(
            rbs[s], out_hbm.at[pl.ds(b0 * _P + ci * _CPX, _CPX)],
            osems[s]).start()

    def out_wait(ci, s):
        pltpu.make_async_copy(
            rbs[s], out_hbm.at[pl.ds(b0 * _P + ci * _CPX, _CPX)],
            osems[s]).wait()

    for s in range(_NSLOT):
        gather(s, s)

    def ring(g, _):
        for s in range(_NSLOT):
            ci = g * _NSLOT + s
            gather_wait(ci, s)
            out_start(ci, s)

            @pl.when(g < _NCH // _NSLOT - 1)
            def _next():
                out_wait(ci, s)
                gather(ci + _NSLOT, s)
        return 0

    lax.fori_loop(0, _NCH // _NSLOT, ring, 0)
    for s in range(_NSLOT):
        out_wait(_NCH - _NSLOT + s, s)


def kernel(inputs, table):
    in_flat = inputs.reshape(-1)
    tab_flat = table.reshape(-1)

    mesh = plsc.VectorSubcoreMesh(core_axis_name="c", subcore_axis_name="s")
    f = functools.partial(
        pl.kernel,
        mesh=mesh,
        out_type=jax.ShapeDtypeStruct((_B * _P, _D), jnp.float32),
        scratch_types=[
            pltpu.VMEM((_BPT * _C * _P,), jnp.int32),
            pltpu.VMEM(((_C * _MAXV) * _D,), jnp.float32),
            pltpu.VMEM((_D, _D), jnp.float32),
            pltpu.VMEM((_BPT * _P,), jnp.int32),
            pltpu.VMEM_SHARED((_CTR, _D), jnp.float32),
        ] + [pltpu.VMEM((_CPX, _D), jnp.float32)] * 8
          + [pltpu.SemaphoreType.DMA] * 18,
        compiler_params=pltpu.CompilerParams(needs_layout_passes=False),
    )(_sc_body)
    out = f(in_flat, tab_flat)
    return out.reshape(_B, _H, _W, _D).transpose(0, 3, 1, 2)
